# baseline (speedup 1.0000x reference)
import numpy as np
import jax
import jax.numpy as jnp
from jax.experimental import pallas as pl
from jax.experimental.pallas import tpu as pltpu

HID = 128
LJ = 7
NF = 6
EPD = 4
AF = LJ * HID
HEADS = 4
BBLK = 1024

_DEG = np.array([2.0, 3.0, 3.0, 3.0, 3.0, 3.0, 2.0])
_DINV = 1.0 / np.sqrt(_DEG)
_A = np.zeros((LJ, LJ))
for _i in range(LJ):
    for _j in range(LJ):
        if abs(_i - _j) <= 1:
            _A[_i, _j] = _DINV[_i] * _DINV[_j]
_A_ROWSUM = _A.sum(axis=1)


def _ln(x, g, b):
    m = jnp.mean(x, axis=-1, keepdims=True)
    d = x - m
    v = jnp.mean(d * d, axis=-1, keepdims=True)
    return d * jax.lax.rsqrt(v + 1e-5) * g + b


def _dot(a, b):
    return jnp.dot(a, b, preferred_element_type=jnp.float32)


def _arm(lq, eps, qe_W, qe_b, g1_W, g1_b, g2_W, g2_b, n1_g, n1_b,
         n2_g, n2_b, gn_g, gn_b, ee_W, ee_b, gat_W, att_s, att_d, gat_b,
         A7, RS7, SD, atsf, JB):
    n = lq.shape[0]

    u = _dot(qe_W, g1_W)
    v = _dot(qe_b, g1_W)
    c = RS7 * v + g1_b
    mu_u = jnp.mean(u, axis=-1, keepdims=True)
    du = u - mu_u
    dc = c - jnp.mean(c, axis=-1, keepdims=True)
    dug = du * n1_g
    dcg = dc * n1_g
    vA = jnp.mean(du * du, axis=-1, keepdims=True)
    vB = jnp.mean(du * dc, axis=-1, keepdims=True)
    vC = jnp.mean(dc * dc, axis=-1, keepdims=True)
    S = _dot(lq, A7)
    slabs = []
    for i in range(LJ):
        s = S[:, i:i + 1]
        var = (vA * s + 2.0 * vB[i:i + 1]) * s + vC[i:i + 1]
        r = jax.lax.rsqrt(var + 1e-5)
        y = (s * dug + dcg[i:i + 1]) * r + n1_b
        slabs.append(jnp.maximum(y, 0.0))

    X1 = jnp.concatenate(slabs, axis=0)
    H = _dot(X1, g2_W)
    Hs = [H[i * n:(i + 1) * n] for i in range(LJ)]
    X2s = []
    for i in range(LJ):
        acc = None
        for j in range(max(0, i - 1), min(LJ, i + 2)):
            t = _A[i, j] * Hs[j]
            acc = t if acc is None else acc + t
        X2s.append(_ln(acc + g2_b, n2_g, n2_b))

    X2 = jnp.concatenate(X2s, axis=0)
    W_mean = 0.25 * (gat_W[:, 0:HID] + gat_W[:, HID:2 * HID]
                     + gat_W[:, 2 * HID:3 * HID] + gat_W[:, 3 * HID:4 * HID])
    base = _dot(X2, W_mean)
    h6 = _dot(X2s[LJ - 1], gat_W)
    Wc = _dot(ee_W, gat_W)
    bc = _dot(ee_b, gat_W)

    head_outs = []
    for k in range(HEADS):
        hk = h6[:, k * HID:(k + 1) * HID]
        atk_s = att_s[k:k + 1, :]
        atk_d = att_d[k:k + 1, :]
        a_s6 = jnp.sum(hk * atk_s, axis=-1, keepdims=True)
        a_d6 = jnp.sum(hk * atk_d, axis=-1, keepdims=True)
        Wck = Wc[:, k * HID:(k + 1) * HID]
        bck = bc[:, k * HID:(k + 1) * HID]
        qsd = [jnp.sum(Wck[d:d + 1, :] * atk_s, axis=-1, keepdims=True)
               for d in range(EPD)]
        bsk = jnp.sum(bck * atk_s, axis=-1, keepdims=True)
        cols = []
        for j in range(NF):
            a = None
            for d in range(EPD):
                t = eps[:, j * EPD + d:j * EPD + d + 1] * qsd[d]
                a = t if a is None else a + t
            cols.append(a + bsk)
        cols.append(a_s6)
        logits = jnp.concatenate(cols, axis=1) + a_d6
        logits = jnp.where(logits > 0, logits, 0.2 * logits)
        mx = jnp.max(logits, axis=-1, keepdims=True)
        e = jnp.exp(logits - mx)
        w = e / jnp.sum(e, axis=-1, keepdims=True)
        eps_eff = None
        for j in range(NF):
            t = w[:, j:j + 1] * eps[:, j * EPD:(j + 1) * EPD]
            eps_eff = t if eps_eff is None else eps_eff + t
        out_k = None
        for d in range(EPD):
            t = eps_eff[:, d:d + 1] * Wck[d:d + 1, :]
            out_k = t if out_k is None else out_k + t
        w_e = jnp.sum(w[:, 0:NF], axis=-1, keepdims=True)
        out_k = out_k + w_e * bck + w[:, NF:NF + 1] * hk
        head_outs.append(out_k)
    out6 = 0.25 * (head_outs[0] + head_outs[1] + head_outs[2] + head_outs[3])

    fins = []
    for i in range(LJ - 1):
        fins.append(_ln(base[i * n:(i + 1) * n] + gat_b, gn_g, gn_b))
    fins.append(_ln(out6 + gat_b, gn_g, gn_b))
    return jnp.concatenate(fins, axis=1)


def _mlp(x, W1, b1, W2, b2, n1_g, n1_b, n2_g, n2_b):
    h = _ln(_dot(x, W1) + b1, n1_g, n1_b)
    h = jnp.maximum(h, 0.0)
    return _ln(_dot(h, W2) + b2, n2_g, n2_b)


def _body(*refs):
    lq_r, rq_r, lef_r, ref_r = refs[0:4]
    lw = [r[...] for r in refs[4:22]]
    rw = [r[...] for r in refs[22:40]]
    l2r = [r[...] for r in refs[40:48]]
    r2l = [r[...] for r in refs[48:56]]
    lf_W, lf_b, rf_W, rf_b = (r[...] for r in refs[56:60])
    A7, RS7, SD_l, atsf_l, SD_r, atsf_r, JB = (r[...] for r in refs[60:67])
    out_r = refs[67]

    lF = _arm(lq_r[...], lef_r[...], *lw, A7, RS7, SD_l, atsf_l, JB)
    rF = _arm(rq_r[...], ref_r[...], *rw, A7, RS7, SD_r, atsf_r, JB)
    lctx = _mlp(lF, *l2r)
    rctx = _mlp(rF, *r2l)
    le = _dot(lF, lf_W[0:AF]) + _dot(rctx, lf_W[AF:AF + HID]) + lf_b
    re = _dot(rF, rf_W[0:AF]) + _dot(lctx, rf_W[AF:AF + HID]) + rf_b
    out_r[:, 0:AF] = le
    out_r[:, AF:2 * AF] = re


def kernel(left_qpos, right_qpos, left_endpose_future, right_endpose_future,
           l_qe_W, l_qe_b, l_g1_W, l_g1_b, l_g2_W, l_g2_b,
           l_n1_g, l_n1_b, l_n2_g, l_n2_b, l_gn_g, l_gn_b,
           l_ee_W, l_ee_b, l_gat_W, l_att_s, l_att_d, l_gat_b,
           r_qe_W, r_qe_b, r_g1_W, r_g1_b, r_g2_W, r_g2_b,
           r_n1_g, r_n1_b, r_n2_g, r_n2_b, r_gn_g, r_gn_b,
           r_ee_W, r_ee_b, r_gat_W, r_att_s, r_att_d, r_gat_b,
           l2r_W1, l2r_b1, l2r_W2, l2r_b2,
           l2r_n1_g, l2r_n1_b, l2r_n2_g, l2r_n2_b,
           r2l_W1, r2l_b1, r2l_W2, r2l_b2,
           r2l_n1_g, r2l_n1_b, r2l_n2_g, r2l_n2_b,
           lf_W, lf_b, rf_W, rf_b):
    b = left_qpos.shape[0]
    row = lambda x: x.reshape(1, -1)
    lef = left_endpose_future.reshape(b, NF * EPD)
    ref2 = right_endpose_future.reshape(b, NF * EPD)

    operands = [
        left_qpos, right_qpos, lef, ref2,
        l_qe_W, row(l_qe_b), l_g1_W, row(l_g1_b), l_g2_W, row(l_g2_b),
        row(l_n1_g), row(l_n1_b), row(l_n2_g), row(l_n2_b),
        row(l_gn_g), row(l_gn_b),
        l_ee_W, row(l_ee_b), l_gat_W, l_att_s, l_att_d, row(l_gat_b),
        r_qe_W, row(r_qe_b), r_g1_W, row(r_g1_b), r_g2_W, row(r_g2_b),
        row(r_n1_g), row(r_n1_b), row(r_n2_g), row(r_n2_b),
        row(r_gn_g), row(r_gn_b),
        r_ee_W, row(r_ee_b), r_gat_W, r_att_s, r_att_d, row(r_gat_b),
        l2r_W1, row(l2r_b1), l2r_W2, row(l2r_b2),
        row(l2r_n1_g), row(l2r_n1_b), row(l2r_n2_g), row(l2r_n2_b),
        r2l_W1, row(r2l_b1), r2l_W2, row(r2l_b2),
        row(r2l_n1_g), row(r2l_n1_b), row(r2l_n2_g), row(r2l_n2_b),
        lf_W, row(lf_b), rf_W, row(rf_b),
    ]

    def sd(att_s, att_d):
        z = jnp.zeros((HEADS * HID, 2 * HEADS), jnp.float32)
        for k in range(HEADS):
            z = z.at[k * HID:(k + 1) * HID, k].set(att_s[k])
            z = z.at[k * HID:(k + 1) * HID, HEADS + k].set(att_d[k])
        return z

    jb = np.zeros((HEADS * HID, HEADS), np.float32)
    for k in range(HEADS):
        jb[k * HID:(k + 1) * HID, k] = 1.0
    operands += [
        jnp.asarray(_A, jnp.float32),
        jnp.asarray(_A_ROWSUM.reshape(LJ, 1), jnp.float32),
        sd(l_att_s, l_att_d), l_att_s.reshape(1, HEADS * HID),
        sd(r_att_s, r_att_d), r_att_s.reshape(1, HEADS * HID),
        jnp.asarray(jb),
    ]

    grid = (b // BBLK,)
    blocked = {0: (BBLK, LJ), 1: (BBLK, LJ), 2: (BBLK, NF * EPD),
               3: (BBLK, NF * EPD)}
    in_specs = []
    for idx, op in enumerate(operands):
        if idx in blocked:
            in_specs.append(pl.BlockSpec(blocked[idx], lambda i: (i, 0)))
        else:
            nd = op.ndim
            in_specs.append(
                pl.BlockSpec(op.shape, lambda i, _n=nd: (0,) * _n))

    return pl.pallas_call(
        _body,
        grid=grid,
        in_specs=in_specs,
        out_specs=pl.BlockSpec((BBLK, 2 * AF), lambda i: (i, 0)),
        out_shape=jax.ShapeDtypeStruct((b, 2 * AF), jnp.float32),
        compiler_params=pltpu.CompilerParams(
            dimension_semantics=("arbitrary",)),
    )(*operands)


# speedup vs baseline: 1.0355x; 1.0355x over previous
import numpy as np
import jax
import jax.numpy as jnp
from jax.experimental import pallas as pl
from jax.experimental.pallas import tpu as pltpu

HID = 128
LJ = 7
NF = 6
EPD = 4
AF = LJ * HID
HEADS = 4
BBLK = 1024

_DEG = np.array([2.0, 3.0, 3.0, 3.0, 3.0, 3.0, 2.0])
_DINV = 1.0 / np.sqrt(_DEG)
_A = np.zeros((LJ, LJ))
for _i in range(LJ):
    for _j in range(LJ):
        if abs(_i - _j) <= 1:
            _A[_i, _j] = _DINV[_i] * _DINV[_j]
_A_ROWSUM = _A.sum(axis=1)


def _ln(x, g, b):
    m = jnp.mean(x, axis=-1, keepdims=True)
    d = x - m
    v = jnp.mean(d * d, axis=-1, keepdims=True)
    return d * jax.lax.rsqrt(v + 1e-5) * g + b


def _dot(a, b):
    return jnp.dot(a, b, preferred_element_type=jnp.float32)


def _arm(lq, eps, qe_W, qe_b, g1_W, g1_b, g2_W, g2_b, n1_g, n1_b,
         n2_g, n2_b, gn_g, gn_b, ee_W, ee_b, gat_W, att_s, att_d, gat_b,
         A7, RS7, SD, atsf, JB):
    n = lq.shape[0]

    u = _dot(qe_W, g1_W)
    v = _dot(qe_b, g1_W)
    slabs = []
    for i in range(LJ):
        s_i = None
        for j in range(max(0, i - 1), min(LJ, i + 2)):
            t = _A[i, j] * lq[:, j:j + 1]
            s_i = t if s_i is None else s_i + t
        x = s_i * u + (_A_ROWSUM[i] * v + g1_b)
        x = _ln(x, n1_g, n1_b)
        slabs.append(jnp.maximum(x, 0.0))

    X1 = jnp.concatenate(slabs, axis=0)
    H = _dot(X1, g2_W)
    Hs = [H[i * n:(i + 1) * n] for i in range(LJ)]
    X2s = []
    for i in range(LJ):
        acc = None
        for j in range(max(0, i - 1), min(LJ, i + 2)):
            t = _A[i, j] * Hs[j]
            acc = t if acc is None else acc + t
        X2s.append(_ln(acc + g2_b, n2_g, n2_b))

    X2 = jnp.concatenate(X2s, axis=0)
    W_mean = 0.25 * (gat_W[:, 0:HID] + gat_W[:, HID:2 * HID]
                     + gat_W[:, 2 * HID:3 * HID] + gat_W[:, 3 * HID:4 * HID])
    base = _dot(X2, W_mean)
    h6 = _dot(X2s[LJ - 1], gat_W)
    Wc = _dot(ee_W, gat_W)
    bc = _dot(ee_b, gat_W)

    asd = _dot(h6, SD)
    ts = Wc * atsf
    qs = _dot(ts, JB)
    bs = _dot(bc * atsf, JB)
    ri = jax.lax.broadcasted_iota(jnp.int32, (NF * EPD, NF), 0)
    ci = jax.lax.broadcasted_iota(jnp.int32, (NF * EPD, NF), 1)
    msk = (ri // EPD == ci).astype(jnp.float32)
    qcols = []
    for k in range(HEADS):
        qk = jnp.concatenate([qs[:, k:k + 1]] * NF, axis=0)
        qcols.append(qk * msk)
    Q24 = jnp.concatenate(qcols, axis=1)
    lg24 = _dot(eps, Q24)

    head_outs = []
    for k in range(HEADS):
        hk = h6[:, k * HID:(k + 1) * HID]
        a_s6 = asd[:, k:k + 1]
        a_d6 = asd[:, HEADS + k:HEADS + k + 1]
        Wck = Wc[:, k * HID:(k + 1) * HID]
        bck = bc[:, k * HID:(k + 1) * HID]
        logits = jnp.concatenate(
            [lg24[:, NF * k:NF * (k + 1)] + bs[:, k:k + 1], a_s6],
            axis=1) + a_d6
        logits = jnp.where(logits > 0, logits, 0.2 * logits)
        mx = jnp.max(logits, axis=-1, keepdims=True)
        e = jnp.exp(logits - mx)
        w = e / jnp.sum(e, axis=-1, keepdims=True)
        eps_eff = None
        for j in range(NF):
            t = w[:, j:j + 1] * eps[:, j * EPD:(j + 1) * EPD]
            eps_eff = t if eps_eff is None else eps_eff + t
        out_k = None
        for d in range(EPD):
            t = eps_eff[:, d:d + 1] * Wck[d:d + 1, :]
            out_k = t if out_k is None else out_k + t
        w_e = jnp.sum(w[:, 0:NF], axis=-1, keepdims=True)
        out_k = out_k + w_e * bck + w[:, NF:NF + 1] * hk
        head_outs.append(out_k)
    out6 = 0.25 * (head_outs[0] + head_outs[1] + head_outs[2] + head_outs[3])

    fins = []
    for i in range(LJ - 1):
        fins.append(_ln(base[i * n:(i + 1) * n] + gat_b, gn_g, gn_b))
    fins.append(_ln(out6 + gat_b, gn_g, gn_b))
    return jnp.concatenate(fins, axis=1)


def _mlp(x, W1, b1, W2, b2, n1_g, n1_b, n2_g, n2_b):
    h = _ln(_dot(x, W1) + b1, n1_g, n1_b)
    h = jnp.maximum(h, 0.0)
    return _ln(_dot(h, W2) + b2, n2_g, n2_b)


def _body(*refs):
    lq_r, rq_r, lef_r, ref_r = refs[0:4]
    lw = [r[...] for r in refs[4:22]]
    rw = [r[...] for r in refs[22:40]]
    l2r = [r[...] for r in refs[40:48]]
    r2l = [r[...] for r in refs[48:56]]
    lf_W, lf_b, rf_W, rf_b = (r[...] for r in refs[56:60])
    A7, RS7, SD_l, atsf_l, SD_r, atsf_r, JB = (r[...] for r in refs[60:67])
    out_r = refs[67]

    lF = _arm(lq_r[...], lef_r[...], *lw, A7, RS7, SD_l, atsf_l, JB)
    rF = _arm(rq_r[...], ref_r[...], *rw, A7, RS7, SD_r, atsf_r, JB)
    lctx = _mlp(lF, *l2r)
    rctx = _mlp(rF, *r2l)
    le = _dot(lF, lf_W[0:AF]) + _dot(rctx, lf_W[AF:AF + HID]) + lf_b
    re = _dot(rF, rf_W[0:AF]) + _dot(lctx, rf_W[AF:AF + HID]) + rf_b
    out_r[:, 0:AF] = le
    out_r[:, AF:2 * AF] = re


def kernel(left_qpos, right_qpos, left_endpose_future, right_endpose_future,
           l_qe_W, l_qe_b, l_g1_W, l_g1_b, l_g2_W, l_g2_b,
           l_n1_g, l_n1_b, l_n2_g, l_n2_b, l_gn_g, l_gn_b,
           l_ee_W, l_ee_b, l_gat_W, l_att_s, l_att_d, l_gat_b,
           r_qe_W, r_qe_b, r_g1_W, r_g1_b, r_g2_W, r_g2_b,
           r_n1_g, r_n1_b, r_n2_g, r_n2_b, r_gn_g, r_gn_b,
           r_ee_W, r_ee_b, r_gat_W, r_att_s, r_att_d, r_gat_b,
           l2r_W1, l2r_b1, l2r_W2, l2r_b2,
           l2r_n1_g, l2r_n1_b, l2r_n2_g, l2r_n2_b,
           r2l_W1, r2l_b1, r2l_W2, r2l_b2,
           r2l_n1_g, r2l_n1_b, r2l_n2_g, r2l_n2_b,
           lf_W, lf_b, rf_W, rf_b):
    b = left_qpos.shape[0]
    row = lambda x: x.reshape(1, -1)
    lef = left_endpose_future.reshape(b, NF * EPD)
    ref2 = right_endpose_future.reshape(b, NF * EPD)

    operands = [
        left_qpos, right_qpos, lef, ref2,
        l_qe_W, row(l_qe_b), l_g1_W, row(l_g1_b), l_g2_W, row(l_g2_b),
        row(l_n1_g), row(l_n1_b), row(l_n2_g), row(l_n2_b),
        row(l_gn_g), row(l_gn_b),
        l_ee_W, row(l_ee_b), l_gat_W, l_att_s, l_att_d, row(l_gat_b),
        r_qe_W, row(r_qe_b), r_g1_W, row(r_g1_b), r_g2_W, row(r_g2_b),
        row(r_n1_g), row(r_n1_b), row(r_n2_g), row(r_n2_b),
        row(r_gn_g), row(r_gn_b),
        r_ee_W, row(r_ee_b), r_gat_W, r_att_s, r_att_d, row(r_gat_b),
        l2r_W1, row(l2r_b1), l2r_W2, row(l2r_b2),
        row(l2r_n1_g), row(l2r_n1_b), row(l2r_n2_g), row(l2r_n2_b),
        r2l_W1, row(r2l_b1), r2l_W2, row(r2l_b2),
        row(r2l_n1_g), row(r2l_n1_b), row(r2l_n2_g), row(r2l_n2_b),
        lf_W, row(lf_b), rf_W, row(rf_b),
    ]

    def sd(att_s, att_d):
        z = jnp.zeros((HEADS * HID, 2 * HEADS), jnp.float32)
        for k in range(HEADS):
            z = z.at[k * HID:(k + 1) * HID, k].set(att_s[k])
            z = z.at[k * HID:(k + 1) * HID, HEADS + k].set(att_d[k])
        return z

    jb = np.zeros((HEADS * HID, HEADS), np.float32)
    for k in range(HEADS):
        jb[k * HID:(k + 1) * HID, k] = 1.0
    operands += [
        jnp.asarray(_A, jnp.float32),
        jnp.asarray(_A_ROWSUM.reshape(LJ, 1), jnp.float32),
        sd(l_att_s, l_att_d), l_att_s.reshape(1, HEADS * HID),
        sd(r_att_s, r_att_d), r_att_s.reshape(1, HEADS * HID),
        jnp.asarray(jb),
    ]

    grid = (b // BBLK,)
    blocked = {0: (BBLK, LJ), 1: (BBLK, LJ), 2: (BBLK, NF * EPD),
               3: (BBLK, NF * EPD)}
    in_specs = []
    for idx, op in enumerate(operands):
        if idx in blocked:
            in_specs.append(pl.BlockSpec(blocked[idx], lambda i: (i, 0)))
        else:
            nd = op.ndim
            in_specs.append(
                pl.BlockSpec(op.shape, lambda i, _n=nd: (0,) * _n))

    return pl.pallas_call(
        _body,
        grid=grid,
        in_specs=in_specs,
        out_specs=pl.BlockSpec((BBLK, 2 * AF), lambda i: (i, 0)),
        out_shape=jax.ShapeDtypeStruct((b, 2 * AF), jnp.float32),
        compiler_params=pltpu.CompilerParams(
            dimension_semantics=("arbitrary",)),
    )(*operands)


# speedup vs baseline: 1.2244x; 1.1825x over previous
import numpy as np
import jax
import jax.numpy as jnp
from jax.experimental import pallas as pl
from jax.experimental.pallas import tpu as pltpu

HID = 128
LJ = 7
NF = 6
EPD = 4
AF = LJ * HID
HEADS = 4
BBLK = 1024

_DEG = np.array([2.0, 3.0, 3.0, 3.0, 3.0, 3.0, 2.0])
_DINV = 1.0 / np.sqrt(_DEG)
_A = np.zeros((LJ, LJ))
for _i in range(LJ):
    for _j in range(LJ):
        if abs(_i - _j) <= 1:
            _A[_i, _j] = _DINV[_i] * _DINV[_j]
_A_ROWSUM = _A.sum(axis=1)


def _ln(x, g, b):
    m = jnp.mean(x, axis=-1, keepdims=True)
    d = x - m
    v = jnp.mean(d * d, axis=-1, keepdims=True)
    return d * jax.lax.rsqrt(v + 1e-5) * g + b


def _dot(a, b):
    return jnp.dot(a, b, preferred_element_type=jnp.float32)


def _arm(lq, eps, qe_W, qe_b, g1_W, g1_b, g2_W, g2_b, n1_g, n1_b,
         n2_g, n2_b, gn_g, gn_b, ee_W, ee_b, gat_W, att_s, att_d, gat_b):
    n = lq.shape[0]

    u = _dot(qe_W, g1_W)
    v = _dot(qe_b, g1_W)
    slabs = []
    for i in range(LJ):
        s_i = None
        for j in range(max(0, i - 1), min(LJ, i + 2)):
            t = _A[i, j] * lq[:, j:j + 1]
            s_i = t if s_i is None else s_i + t
        x = s_i * u + (_A_ROWSUM[i] * v + g1_b)
        x = _ln(x, n1_g, n1_b)
        slabs.append(jnp.maximum(x, 0.0))

    X1 = jnp.concatenate(slabs, axis=0)
    H = _dot(X1, g2_W)
    Hs = [H[i * n:(i + 1) * n] for i in range(LJ)]
    X2s = []
    for i in range(LJ):
        acc = None
        for j in range(max(0, i - 1), min(LJ, i + 2)):
            t = _A[i, j] * Hs[j]
            acc = t if acc is None else acc + t
        X2s.append(_ln(acc + g2_b, n2_g, n2_b))

    X2 = jnp.concatenate(X2s, axis=0)
    W_mean = 0.25 * (gat_W[:, 0:HID] + gat_W[:, HID:2 * HID]
                     + gat_W[:, 2 * HID:3 * HID] + gat_W[:, 3 * HID:4 * HID])
    base = _dot(X2, W_mean)
    h6 = _dot(X2s[LJ - 1], gat_W)
    Wc = _dot(ee_W, gat_W)
    bc = _dot(ee_b, gat_W)

    head_outs = []
    for k in range(HEADS):
        hk = h6[:, k * HID:(k + 1) * HID]
        atk_s = att_s[k:k + 1, :]
        atk_d = att_d[k:k + 1, :]
        a_s6 = jnp.sum(hk * atk_s, axis=-1, keepdims=True)
        a_d6 = jnp.sum(hk * atk_d, axis=-1, keepdims=True)
        Wck = Wc[:, k * HID:(k + 1) * HID]
        bck = bc[:, k * HID:(k + 1) * HID]
        qsd = [jnp.sum(Wck[d:d + 1, :] * atk_s, axis=-1, keepdims=True)
               for d in range(EPD)]
        bsk = jnp.sum(bck * atk_s, axis=-1, keepdims=True)
        cols = []
        for j in range(NF):
            a = None
            for d in range(EPD):
                t = eps[:, j * EPD + d:j * EPD + d + 1] * qsd[d]
                a = t if a is None else a + t
            cols.append(a + bsk)
        cols.append(a_s6)
        logits = jnp.concatenate(cols, axis=1) + a_d6
        logits = jnp.where(logits > 0, logits, 0.2 * logits)
        mx = jnp.max(logits, axis=-1, keepdims=True)
        e = jnp.exp(logits - mx)
        w = e / jnp.sum(e, axis=-1, keepdims=True)
        eps_eff = None
        for j in range(NF):
            t = w[:, j:j + 1] * eps[:, j * EPD:(j + 1) * EPD]
            eps_eff = t if eps_eff is None else eps_eff + t
        out_k = None
        for d in range(EPD):
            t = eps_eff[:, d:d + 1] * Wck[d:d + 1, :]
            out_k = t if out_k is None else out_k + t
        w_e = jnp.sum(w[:, 0:NF], axis=-1, keepdims=True)
        out_k = out_k + w_e * bck + w[:, NF:NF + 1] * hk
        head_outs.append(out_k)
    out6 = 0.25 * (head_outs[0] + head_outs[1] + head_outs[2] + head_outs[3])

    fins = []
    for i in range(LJ - 1):
        fins.append(_ln(base[i * n:(i + 1) * n] + gat_b, gn_g, gn_b))
    fins.append(_ln(out6 + gat_b, gn_g, gn_b))
    return jnp.concatenate(fins, axis=1)


def _mlp(x, W1, b1, W2, b2, n1_g, n1_b, n2_g, n2_b):
    h = _ln(_dot(x, W1) + b1, n1_g, n1_b)
    h = jnp.maximum(h, 0.0)
    return _ln(_dot(h, W2) + b2, n2_g, n2_b)


def _body(*refs):
    lq_r, rq_r, lef_r, ref_r = refs[0:4]
    lw = [r[...] for r in refs[4:22]]
    rw = [r[...] for r in refs[22:40]]
    l2r = [r[...] for r in refs[40:48]]
    r2l = [r[...] for r in refs[48:56]]
    lf_W, lf_b, rf_W, rf_b = (r[...] for r in refs[56:60])
    out_r = refs[60]

    lF = _arm(lq_r[...], lef_r[...], *lw)
    rF = _arm(rq_r[...], ref_r[...], *rw)
    lctx = _mlp(lF, *l2r)
    rctx = _mlp(rF, *r2l)
    le = _dot(lF, lf_W[0:AF]) + _dot(rctx, lf_W[AF:AF + HID]) + lf_b
    re = _dot(rF, rf_W[0:AF]) + _dot(lctx, rf_W[AF:AF + HID]) + rf_b
    out_r[:, 0:AF] = le
    out_r[:, AF:2 * AF] = re


def kernel(left_qpos, right_qpos, left_endpose_future, right_endpose_future,
           l_qe_W, l_qe_b, l_g1_W, l_g1_b, l_g2_W, l_g2_b,
           l_n1_g, l_n1_b, l_n2_g, l_n2_b, l_gn_g, l_gn_b,
           l_ee_W, l_ee_b, l_gat_W, l_att_s, l_att_d, l_gat_b,
           r_qe_W, r_qe_b, r_g1_W, r_g1_b, r_g2_W, r_g2_b,
           r_n1_g, r_n1_b, r_n2_g, r_n2_b, r_gn_g, r_gn_b,
           r_ee_W, r_ee_b, r_gat_W, r_att_s, r_att_d, r_gat_b,
           l2r_W1, l2r_b1, l2r_W2, l2r_b2,
           l2r_n1_g, l2r_n1_b, l2r_n2_g, l2r_n2_b,
           r2l_W1, r2l_b1, r2l_W2, r2l_b2,
           r2l_n1_g, r2l_n1_b, r2l_n2_g, r2l_n2_b,
           lf_W, lf_b, rf_W, rf_b):
    b = left_qpos.shape[0]
    row = lambda x: x.reshape(1, -1)
    lef = left_endpose_future.reshape(b, NF * EPD)
    ref2 = right_endpose_future.reshape(b, NF * EPD)

    operands = [
        left_qpos, right_qpos, lef, ref2,
        l_qe_W, row(l_qe_b), l_g1_W, row(l_g1_b), l_g2_W, row(l_g2_b),
        row(l_n1_g), row(l_n1_b), row(l_n2_g), row(l_n2_b),
        row(l_gn_g), row(l_gn_b),
        l_ee_W, row(l_ee_b), l_gat_W, l_att_s, l_att_d, row(l_gat_b),
        r_qe_W, row(r_qe_b), r_g1_W, row(r_g1_b), r_g2_W, row(r_g2_b),
        row(r_n1_g), row(r_n1_b), row(r_n2_g), row(r_n2_b),
        row(r_gn_g), row(r_gn_b),
        r_ee_W, row(r_ee_b), r_gat_W, r_att_s, r_att_d, row(r_gat_b),
        l2r_W1, row(l2r_b1), l2r_W2, row(l2r_b2),
        row(l2r_n1_g), row(l2r_n1_b), row(l2r_n2_g), row(l2r_n2_b),
        r2l_W1, row(r2l_b1), r2l_W2, row(r2l_b2),
        row(r2l_n1_g), row(r2l_n1_b), row(r2l_n2_g), row(r2l_n2_b),
        lf_W, row(lf_b), rf_W, row(rf_b),
    ]

    grid = (b // BBLK,)
    blocked = {0: (BBLK, LJ), 1: (BBLK, LJ), 2: (BBLK, NF * EPD),
               3: (BBLK, NF * EPD)}
    in_specs = []
    for idx, op in enumerate(operands):
        if idx in blocked:
            in_specs.append(pl.BlockSpec(blocked[idx], lambda i: (i, 0)))
        else:
            nd = op.ndim
            in_specs.append(
                pl.BlockSpec(op.shape, lambda i, _n=nd: (0,) * _n))

    return pl.pallas_call(
        _body,
        grid=grid,
        in_specs=in_specs,
        out_specs=pl.BlockSpec((BBLK, 2 * AF), lambda i: (i, 0)),
        out_shape=jax.ShapeDtypeStruct((b, 2 * AF), jnp.float32),
        compiler_params=pltpu.CompilerParams(
            dimension_semantics=("arbitrary",)),
    )(*operands)
